# baseline (speedup 1.0000x reference)
import functools

import numpy as np
import jax
import jax.numpy as jnp
from jax import lax
from jax.experimental import pallas as pl
from jax.experimental.pallas import tpu as pltpu
from jax.experimental.pallas import tpu_sc as plsc

NUM_LEVELS = 16
MIN_RES = 128
MAX_RES = 2048
LOG2_HASHMAP_SIZE = 19
HSIZE = 1 << LOG2_HASHMAP_SIZE
F = 2
_b = np.exp((np.log(MAX_RES) - np.log(MIN_RES)) / (NUM_LEVELS - 1))
_RES = [int(np.floor(MIN_RES * _b ** lvl)) for lvl in range(NUM_LEVELS)]

P1 = 73856093
P2 = 19349663
P3 = 83492791

NC = 2
NS = 16
L = 16
NW = NC * NS
LVL_WORDS = HSIZE * F


@functools.lru_cache(maxsize=None)
def _build(N):
    NPW = N // NW
    C = 512
    NCH = NPW // C
    G = C // L
    CI = 16 * C
    STG = LVL_WORDS // NS

    mesh = plsc.VectorSubcoreMesh(core_axis_name="c", subcore_axis_name="s")

    @functools.partial(
        pl.kernel,
        mesh=mesh,
        out_type=jax.ShapeDtypeStruct((N * NUM_LEVELS * F,), jnp.float32),
        scratch_types=[
            pltpu.VMEM_SHARED((LVL_WORDS,), jnp.float32),
            pltpu.VMEM((L,), jnp.float32),
            pltpu.VMEM((2, C), jnp.float32),
            pltpu.VMEM((2, C), jnp.float32),
            pltpu.VMEM((2, C), jnp.float32),
            pltpu.VMEM((2, C), jnp.float32),
            pltpu.VMEM((2, C), jnp.float32),
            pltpu.VMEM((2, C), jnp.float32),
            pltpu.VMEM((CI,), jnp.int32),
            pltpu.VMEM((CI,), jnp.int32),
            pltpu.VMEM((CI,), jnp.float32),
            pltpu.VMEM((CI,), jnp.float32),
            pltpu.VMEM((2 * C,), jnp.float32),
            pltpu.VMEM((2 * C,), jnp.int32),
            pltpu.SemaphoreType.DMA,
            pltpu.SemaphoreType.DMA,
            pltpu.SemaphoreType.DMA,
        ],
    )
    def hash_enc(xs, ys, zs, tbl, rconst, out, spm, rvb, xb, yb, zb, wxb, wyb,
                 wzb, idxb0, idxb1, rowsb0, rowsb1, valb, oixb, gsem0, gsem1,
                 ssem):
        sid = lax.axis_index("s")
        wid = sid * NC + lax.axis_index("c")
        lane = lax.iota(jnp.int32, L)
        mask = jnp.int32(HSIZE - 1)
        low7 = jnp.int32(127)
        wbase = wid * jnp.int32(NPW)

        def level_body(l, carry):
            plsc.subcore_barrier()
            lbase = l * jnp.int32(LVL_WORDS)
            pltpu.sync_copy(tbl.at[pl.ds(lbase + sid * jnp.int32(STG), STG)],
                            spm.at[pl.ds(sid * jnp.int32(STG), STG)])
            pltpu.sync_copy(rconst.at[pl.ds(l * jnp.int32(L), L)], rvb)
            plsc.subcore_barrier()
            resv = rvb[pl.ds(jnp.int32(0), L)]
            resm1 = resv.astype(jnp.int32) - jnp.int32(1)
            obase0 = l * jnp.int32(2)

            def fire(k, par):
                idxb = idxb0 if par == 0 else idxb1
                sem = gsem0 if par == 0 else gsem1
                rowsb = rowsb0 if par == 0 else rowsb1
                pbase = wbase + k * jnp.int32(C)
                pj = jnp.int32(par)
                pltpu.sync_copy(xs.at[pl.ds(pbase, C)], xb.at[pj])
                pltpu.sync_copy(ys.at[pl.ds(pbase, C)], yb.at[pj])
                pltpu.sync_copy(zs.at[pl.ds(pbase, C)], zb.at[pj])

                def idx_body(g, c2):
                    p = g * jnp.int32(L)
                    xv = xb[pj, pl.ds(p, L)]
                    yv = yb[pj, pl.ds(p, L)]
                    zv = zb[pj, pl.ds(p, L)]
                    sx = xv * resv
                    sy = yv * resv
                    sz = zv * resv
                    tx = sx.astype(jnp.int32)
                    ty = sy.astype(jnp.int32)
                    tz = sz.astype(jnp.int32)
                    wxb[pj, pl.ds(p, L)] = sx - tx.astype(jnp.float32)
                    wyb[pj, pl.ds(p, L)] = sy - ty.astype(jnp.float32)
                    wzb[pj, pl.ds(p, L)] = sz - tz.astype(jnp.float32)
                    ix = jnp.minimum(tx, resm1)
                    iy = jnp.minimum(ty, resm1)
                    iz = jnp.minimum(tz, resm1)
                    hx0 = ix * jnp.int32(P1)
                    hx1 = hx0 + jnp.int32(P1)
                    hy0 = iy * jnp.int32(P2)
                    hy1 = hy0 + jnp.int32(P2)
                    hz0 = iz * jnp.int32(P3)
                    hz1 = hz0 + jnp.int32(P3)
                    c = 0
                    for hx in (hx0, hx1):
                        hxy0 = hx ^ hy0
                        hxy1 = hx ^ hy1
                        for hxy in (hxy0, hxy1):
                            for hz in (hz0, hz1):
                                t = (hxy ^ hz) & mask
                                a0 = ((t >> jnp.int32(7)) << jnp.int32(8)) \
                                    | (t & low7)
                                idxb[pl.ds(jnp.int32(2 * c * C) + p, L)] = a0
                                idxb[pl.ds(jnp.int32((2 * c + 1) * C) + p,
                                           L)] = a0 + jnp.int32(128)
                                c += 1
                    return c2

                lax.fori_loop(jnp.int32(0), jnp.int32(G), idx_body,
                              jnp.int32(0), unroll=False)
                pltpu.async_copy(spm.at[idxb], rowsb, sem)

            def drain(k, par):
                idxb = idxb0 if par == 0 else idxb1
                sem = gsem0 if par == 0 else gsem1
                rowsb = rowsb0 if par == 0 else rowsb1
                pbase = wbase + k * jnp.int32(C)
                pj = jnp.int32(par)
                pltpu.make_async_copy(spm.at[idxb], rowsb, sem).wait()

                def interp_body(g, c2):
                    p = g * jnp.int32(L)
                    wx = wxb[pj, pl.ds(p, L)]
                    wy = wyb[pj, pl.ds(p, L)]
                    wz = wzb[pj, pl.ds(p, L)]
                    umx = 1.0 - wx
                    umy = 1.0 - wy
                    umz = 1.0 - wz
                    opos = (pbase + p + lane) * jnp.int32(NUM_LEVELS * F) \
                        + obase0
                    for feat in range(2):
                        f = [rowsb[pl.ds(jnp.int32((2 * c + feat) * C) + p,
                                         L)]
                             for c in range(8)]
                        fx00 = f[0] * umx + f[4] * wx
                        fx01 = f[1] * umx + f[5] * wx
                        fx10 = f[2] * umx + f[6] * wx
                        fx11 = f[3] * umx + f[7] * wx
                        fxy0 = fx00 * umy + fx10 * wy
                        fxy1 = fx01 * umy + fx11 * wy
                        val = fxy0 * umz + fxy1 * wz
                        valb[pl.ds(jnp.int32(feat * C) + p, L)] = val
                        oixb[pl.ds(jnp.int32(feat * C) + p, L)] = \
                            opos + jnp.int32(feat)
                    return c2

                lax.fori_loop(jnp.int32(0), jnp.int32(G), interp_body,
                              jnp.int32(0), unroll=False)
                pltpu.async_copy(valb, out.at[oixb], ssem).wait()

            fire(jnp.int32(0), 0)

            def chunk_body(j, carry):
                k = j * jnp.int32(2)
                fire(k + jnp.int32(1), 1)
                drain(k, 0)
                fire(k + jnp.int32(2), 0)
                drain(k + jnp.int32(1), 1)
                return carry

            lax.fori_loop(jnp.int32(0), jnp.int32(NCH // 2 - 1), chunk_body,
                          jnp.int32(0), unroll=False)
            klast = jnp.int32(NCH - 2)
            fire(klast + jnp.int32(1), 1)
            drain(klast, 0)
            drain(klast + jnp.int32(1), 1)
            return carry

        lax.fori_loop(jnp.int32(0), jnp.int32(NUM_LEVELS), level_body,
                      jnp.int32(0), unroll=False)

    return hash_enc


def kernel(x, tables):
    shape = x.shape
    xf = x.reshape(-1, 3)
    N = xf.shape[0]
    tbl = tables.reshape(NUM_LEVELS, HSIZE // 128, 128, F) \
        .transpose(0, 1, 3, 2).reshape(-1)
    rconst = jnp.asarray(
        np.repeat(np.array(_RES, np.float32), L))
    out = _build(N)(xf[:, 0], xf[:, 1], xf[:, 2], tbl, rconst)
    return out.reshape(*shape[:-1], NUM_LEVELS * F)


# speedup vs baseline: 8.3854x; 8.3854x over previous
import functools

import numpy as np
import jax
import jax.numpy as jnp
from jax import lax
from jax.experimental import pallas as pl
from jax.experimental.pallas import tpu as pltpu
from jax.experimental.pallas import tpu_sc as plsc

NUM_LEVELS = 16
MIN_RES = 128
MAX_RES = 2048
LOG2_HASHMAP_SIZE = 19
HSIZE = 1 << LOG2_HASHMAP_SIZE
F = 2
_b = np.exp((np.log(MAX_RES) - np.log(MIN_RES)) / (NUM_LEVELS - 1))
_RES = [int(np.floor(MIN_RES * _b ** lvl)) for lvl in range(NUM_LEVELS)]

P1 = 73856093
P2 = 19349663
P3 = 83492791

NC = 2
NS = 16
L = 16
NW = NC * NS
LVL_WORDS = HSIZE * F


@functools.lru_cache(maxsize=None)
def _build(N):
    NPW = N // NW
    C = 512
    NCH = NPW // C
    G = C // L
    CI = 16 * C
    STG = LVL_WORDS // NS

    mesh = plsc.VectorSubcoreMesh(core_axis_name="c", subcore_axis_name="s")

    @functools.partial(
        pl.kernel,
        mesh=mesh,
        out_type=jax.ShapeDtypeStruct((NUM_LEVELS * F, N), jnp.float32),
        scratch_types=[
            pltpu.VMEM_SHARED((LVL_WORDS,), jnp.float32),
            pltpu.VMEM((L,), jnp.float32),
            pltpu.VMEM((2, C), jnp.float32),
            pltpu.VMEM((2, C), jnp.float32),
            pltpu.VMEM((2, C), jnp.float32),
            pltpu.VMEM((2, C), jnp.float32),
            pltpu.VMEM((2, C), jnp.float32),
            pltpu.VMEM((2, C), jnp.float32),
            pltpu.VMEM((CI,), jnp.int32),
            pltpu.VMEM((CI,), jnp.int32),
            pltpu.VMEM((CI,), jnp.float32),
            pltpu.VMEM((CI,), jnp.float32),
            pltpu.VMEM((2 * C,), jnp.float32),
            pltpu.SemaphoreType.DMA,
            pltpu.SemaphoreType.DMA,
        ],
    )
    def hash_enc(xs, ys, zs, tbl, rconst, out, spm, rvb, xb, yb, zb, wxb,
                 wyb, wzb, idxb0, idxb1, rowsb0, rowsb1, valb, gsem0, gsem1):
        sid = lax.axis_index("s")
        wid = sid * NC + lax.axis_index("c")
        lane = lax.iota(jnp.int32, L)
        mask = jnp.int32(HSIZE - 1)
        low7 = jnp.int32(127)
        wbase = wid * jnp.int32(NPW)

        def level_body(l, carry):
            plsc.subcore_barrier()
            lbase = l * jnp.int32(LVL_WORDS)
            pltpu.sync_copy(tbl.at[pl.ds(lbase + sid * jnp.int32(STG), STG)],
                            spm.at[pl.ds(sid * jnp.int32(STG), STG)])
            pltpu.sync_copy(rconst.at[pl.ds(l * jnp.int32(L), L)], rvb)
            plsc.subcore_barrier()
            resv = rvb[pl.ds(jnp.int32(0), L)]
            resm1 = resv.astype(jnp.int32) - jnp.int32(1)
            obase0 = l * jnp.int32(2)

            def fire(k, par):
                idxb = idxb0 if par == 0 else idxb1
                sem = gsem0 if par == 0 else gsem1
                rowsb = rowsb0 if par == 0 else rowsb1
                pbase = wbase + k * jnp.int32(C)
                pj = jnp.int32(par)
                pltpu.sync_copy(xs.at[pl.ds(pbase, C)], xb.at[pj])
                pltpu.sync_copy(ys.at[pl.ds(pbase, C)], yb.at[pj])
                pltpu.sync_copy(zs.at[pl.ds(pbase, C)], zb.at[pj])

                def idx_body(g, c2):
                    p = g * jnp.int32(L)
                    xv = xb[pj, pl.ds(p, L)]
                    yv = yb[pj, pl.ds(p, L)]
                    zv = zb[pj, pl.ds(p, L)]
                    sx = xv * resv
                    sy = yv * resv
                    sz = zv * resv
                    tx = sx.astype(jnp.int32)
                    ty = sy.astype(jnp.int32)
                    tz = sz.astype(jnp.int32)
                    wxb[pj, pl.ds(p, L)] = sx - tx.astype(jnp.float32)
                    wyb[pj, pl.ds(p, L)] = sy - ty.astype(jnp.float32)
                    wzb[pj, pl.ds(p, L)] = sz - tz.astype(jnp.float32)
                    ix = jnp.minimum(tx, resm1)
                    iy = jnp.minimum(ty, resm1)
                    iz = jnp.minimum(tz, resm1)
                    hx0 = ix * jnp.int32(P1)
                    hx1 = hx0 + jnp.int32(P1)
                    hy0 = iy * jnp.int32(P2)
                    hy1 = hy0 + jnp.int32(P2)
                    hz0 = iz * jnp.int32(P3)
                    hz1 = hz0 + jnp.int32(P3)
                    c = 0
                    for hx in (hx0, hx1):
                        hxy0 = hx ^ hy0
                        hxy1 = hx ^ hy1
                        for hxy in (hxy0, hxy1):
                            for hz in (hz0, hz1):
                                t = (hxy ^ hz) & mask
                                a0 = ((t >> jnp.int32(7)) << jnp.int32(8)) \
                                    | (t & low7)
                                idxb[pl.ds(jnp.int32(2 * c * C) + p, L)] = a0
                                idxb[pl.ds(jnp.int32((2 * c + 1) * C) + p,
                                           L)] = a0 + jnp.int32(128)
                                c += 1
                    return c2

                lax.fori_loop(jnp.int32(0), jnp.int32(G), idx_body,
                              jnp.int32(0), unroll=False)
                pltpu.async_copy(spm.at[idxb], rowsb, sem)

            def drain(k, par):
                idxb = idxb0 if par == 0 else idxb1
                sem = gsem0 if par == 0 else gsem1
                rowsb = rowsb0 if par == 0 else rowsb1
                pbase = wbase + k * jnp.int32(C)
                pj = jnp.int32(par)
                pltpu.make_async_copy(spm.at[idxb], rowsb, sem).wait()

                def interp_body(g, c2):
                    p = g * jnp.int32(L)
                    wx = wxb[pj, pl.ds(p, L)]
                    wy = wyb[pj, pl.ds(p, L)]
                    wz = wzb[pj, pl.ds(p, L)]
                    umx = 1.0 - wx
                    umy = 1.0 - wy
                    umz = 1.0 - wz
                    for feat in range(2):
                        f = [rowsb[pl.ds(jnp.int32((2 * c + feat) * C) + p,
                                         L)]
                             for c in range(8)]
                        fx00 = f[0] * umx + f[4] * wx
                        fx01 = f[1] * umx + f[5] * wx
                        fx10 = f[2] * umx + f[6] * wx
                        fx11 = f[3] * umx + f[7] * wx
                        fxy0 = fx00 * umy + fx10 * wy
                        fxy1 = fx01 * umy + fx11 * wy
                        val = fxy0 * umz + fxy1 * wz
                        valb[pl.ds(jnp.int32(feat * C) + p, L)] = val
                    return c2

                lax.fori_loop(jnp.int32(0), jnp.int32(G), interp_body,
                              jnp.int32(0), unroll=False)
                pltpu.sync_copy(valb.at[pl.ds(jnp.int32(0), C)],
                                out.at[obase0, pl.ds(pbase, C)])
                pltpu.sync_copy(valb.at[pl.ds(jnp.int32(C), C)],
                                out.at[obase0 + jnp.int32(1),
                                       pl.ds(pbase, C)])

            fire(jnp.int32(0), 0)

            def chunk_body(j, carry):
                k = j * jnp.int32(2)
                fire(k + jnp.int32(1), 1)
                drain(k, 0)
                fire(k + jnp.int32(2), 0)
                drain(k + jnp.int32(1), 1)
                return carry

            lax.fori_loop(jnp.int32(0), jnp.int32(NCH // 2 - 1), chunk_body,
                          jnp.int32(0), unroll=False)
            klast = jnp.int32(NCH - 2)
            fire(klast + jnp.int32(1), 1)
            drain(klast, 0)
            drain(klast + jnp.int32(1), 1)
            return carry

        lax.fori_loop(jnp.int32(0), jnp.int32(NUM_LEVELS), level_body,
                      jnp.int32(0), unroll=False)

    return hash_enc


def _tbody(in_ref, out_ref):
    out_ref[...] = in_ref[...].T


@functools.lru_cache(maxsize=None)
def _transposer(n):
    B = 2048
    return pl.pallas_call(
        _tbody,
        grid=(n // B,),
        in_specs=[pl.BlockSpec((NUM_LEVELS * F, B),
                               lambda i: (jnp.int32(0), i))],
        out_specs=pl.BlockSpec((B, NUM_LEVELS * F),
                              lambda i: (i, jnp.int32(0))),
        out_shape=jax.ShapeDtypeStruct((n, NUM_LEVELS * F), jnp.float32),
    )


def kernel(x, tables):
    shape = x.shape
    xf = x.reshape(-1, 3)
    N = xf.shape[0]
    tbl = tables.reshape(NUM_LEVELS, HSIZE // 128, 128, F) \
        .transpose(0, 1, 3, 2).reshape(-1)
    rconst = jnp.asarray(
        np.repeat(np.array(_RES, np.float32), L))
    olm = _build(N)(xf[:, 0], xf[:, 1], xf[:, 2], tbl, rconst)
    out = _transposer(N)(olm)
    return out.reshape(*shape[:-1], NUM_LEVELS * F)


# speedup vs baseline: 10.4250x; 1.2432x over previous
import functools

import numpy as np
import jax
import jax.numpy as jnp
from jax import lax
from jax.experimental import pallas as pl
from jax.experimental.pallas import tpu as pltpu
from jax.experimental.pallas import tpu_sc as plsc

NUM_LEVELS = 16
MIN_RES = 128
MAX_RES = 2048
LOG2_HASHMAP_SIZE = 19
HSIZE = 1 << LOG2_HASHMAP_SIZE
F = 2
_b = np.exp((np.log(MAX_RES) - np.log(MIN_RES)) / (NUM_LEVELS - 1))
_RES = [int(np.floor(MIN_RES * _b ** lvl)) for lvl in range(NUM_LEVELS)]

P1 = 73856093
P2 = 19349663
P3 = 83492791

NC = 2
NS = 16
L = 16
NW = NC * NS
LVL_WORDS = HSIZE * F


@functools.lru_cache(maxsize=None)
def _build(N):
    NPW = N // NW
    C = 512
    NCH = NPW // C
    G = C // L
    CI = 16 * C
    STG = LVL_WORDS // NS

    mesh = plsc.VectorSubcoreMesh(core_axis_name="c", subcore_axis_name="s")

    @functools.partial(
        pl.kernel,
        mesh=mesh,
        out_type=jax.ShapeDtypeStruct((NUM_LEVELS * F, N), jnp.float32),
        scratch_types=[
            pltpu.VMEM_SHARED((LVL_WORDS,), jnp.float32),
            pltpu.VMEM((L,), jnp.float32),
            pltpu.VMEM((NPW,), jnp.float32),
            pltpu.VMEM((NPW,), jnp.float32),
            pltpu.VMEM((NPW,), jnp.float32),
            pltpu.VMEM((2, C), jnp.float32),
            pltpu.VMEM((2, C), jnp.float32),
            pltpu.VMEM((2, C), jnp.float32),
            pltpu.VMEM((CI,), jnp.int32),
            pltpu.VMEM((CI,), jnp.int32),
            pltpu.VMEM((CI,), jnp.float32),
            pltpu.VMEM((CI,), jnp.float32),
            pltpu.VMEM((2 * C,), jnp.float32),
            pltpu.SemaphoreType.DMA,
            pltpu.SemaphoreType.DMA,
        ],
    )
    def hash_enc(xs, ys, zs, tbl, rconst, out, spm, rvb, xb, yb, zb, wxb,
                 wyb, wzb, idxb0, idxb1, rowsb0, rowsb1, valb, gsem0, gsem1):
        sid = lax.axis_index("s")
        wid = sid * NC + lax.axis_index("c")
        lane = lax.iota(jnp.int32, L)
        mask = jnp.int32(HSIZE - 1)
        low7 = jnp.int32(127)
        wbase = wid * jnp.int32(NPW)
        pltpu.sync_copy(xs.at[pl.ds(wbase, NPW)], xb)
        pltpu.sync_copy(ys.at[pl.ds(wbase, NPW)], yb)
        pltpu.sync_copy(zs.at[pl.ds(wbase, NPW)], zb)

        def level_body(l, carry):
            plsc.subcore_barrier()
            lbase = l * jnp.int32(LVL_WORDS)
            pltpu.sync_copy(tbl.at[pl.ds(lbase + sid * jnp.int32(STG), STG)],
                            spm.at[pl.ds(sid * jnp.int32(STG), STG)])
            pltpu.sync_copy(rconst.at[pl.ds(l * jnp.int32(L), L)], rvb)
            plsc.subcore_barrier()
            resv = rvb[pl.ds(jnp.int32(0), L)]
            resm1 = resv.astype(jnp.int32) - jnp.int32(1)
            obase0 = l * jnp.int32(2)

            def fire(k, par):
                idxb = idxb0 if par == 0 else idxb1
                sem = gsem0 if par == 0 else gsem1
                rowsb = rowsb0 if par == 0 else rowsb1
                lbase2 = k * jnp.int32(C)
                pj = jnp.int32(par)

                def idx_body(g, c2):
                    p = g * jnp.int32(L)
                    xv = xb[pl.ds(lbase2 + p, L)]
                    yv = yb[pl.ds(lbase2 + p, L)]
                    zv = zb[pl.ds(lbase2 + p, L)]
                    sx = xv * resv
                    sy = yv * resv
                    sz = zv * resv
                    tx = sx.astype(jnp.int32)
                    ty = sy.astype(jnp.int32)
                    tz = sz.astype(jnp.int32)
                    wxb[pj, pl.ds(p, L)] = sx - tx.astype(jnp.float32)
                    wyb[pj, pl.ds(p, L)] = sy - ty.astype(jnp.float32)
                    wzb[pj, pl.ds(p, L)] = sz - tz.astype(jnp.float32)
                    ix = jnp.minimum(tx, resm1)
                    iy = jnp.minimum(ty, resm1)
                    iz = jnp.minimum(tz, resm1)
                    hx0 = ix * jnp.int32(P1)
                    hx1 = hx0 + jnp.int32(P1)
                    hy0 = iy * jnp.int32(P2)
                    hy1 = hy0 + jnp.int32(P2)
                    hz0 = iz * jnp.int32(P3)
                    hz1 = hz0 + jnp.int32(P3)
                    c = 0
                    for hx in (hx0, hx1):
                        hxy0 = hx ^ hy0
                        hxy1 = hx ^ hy1
                        for hxy in (hxy0, hxy1):
                            for hz in (hz0, hz1):
                                t = (hxy ^ hz) & mask
                                a0 = ((t >> jnp.int32(7)) << jnp.int32(8)) \
                                    | (t & low7)
                                idxb[pl.ds(jnp.int32(2 * c * C) + p, L)] = a0
                                idxb[pl.ds(jnp.int32((2 * c + 1) * C) + p,
                                           L)] = a0 + jnp.int32(128)
                                c += 1
                    return c2

                lax.fori_loop(jnp.int32(0), jnp.int32(G), idx_body,
                              jnp.int32(0), unroll=False)
                pltpu.async_copy(spm.at[idxb], rowsb, sem)

            def drain(k, par):
                idxb = idxb0 if par == 0 else idxb1
                sem = gsem0 if par == 0 else gsem1
                rowsb = rowsb0 if par == 0 else rowsb1
                pbase = wbase + k * jnp.int32(C)
                pj = jnp.int32(par)
                pltpu.make_async_copy(spm.at[idxb], rowsb, sem).wait()

                def interp_body(g, c2):
                    p = g * jnp.int32(L)
                    wx = wxb[pj, pl.ds(p, L)]
                    wy = wyb[pj, pl.ds(p, L)]
                    wz = wzb[pj, pl.ds(p, L)]
                    umx = 1.0 - wx
                    umy = 1.0 - wy
                    umz = 1.0 - wz
                    for feat in range(2):
                        f = [rowsb[pl.ds(jnp.int32((2 * c + feat) * C) + p,
                                         L)]
                             for c in range(8)]
                        fx00 = f[0] * umx + f[4] * wx
                        fx01 = f[1] * umx + f[5] * wx
                        fx10 = f[2] * umx + f[6] * wx
                        fx11 = f[3] * umx + f[7] * wx
                        fxy0 = fx00 * umy + fx10 * wy
                        fxy1 = fx01 * umy + fx11 * wy
                        val = fxy0 * umz + fxy1 * wz
                        valb[pl.ds(jnp.int32(feat * C) + p, L)] = val
                    return c2

                lax.fori_loop(jnp.int32(0), jnp.int32(G), interp_body,
                              jnp.int32(0), unroll=False)
                pltpu.sync_copy(valb.at[pl.ds(jnp.int32(0), C)],
                                out.at[obase0, pl.ds(pbase, C)])
                pltpu.sync_copy(valb.at[pl.ds(jnp.int32(C), C)],
                                out.at[obase0 + jnp.int32(1),
                                       pl.ds(pbase, C)])

            fire(jnp.int32(0), 0)

            def chunk_body(j, carry):
                k = j * jnp.int32(2)
                fire(k + jnp.int32(1), 1)
                drain(k, 0)
                fire(k + jnp.int32(2), 0)
                drain(k + jnp.int32(1), 1)
                return carry

            lax.fori_loop(jnp.int32(0), jnp.int32(NCH // 2 - 1), chunk_body,
                          jnp.int32(0), unroll=False)
            klast = jnp.int32(NCH - 2)
            fire(klast + jnp.int32(1), 1)
            drain(klast, 0)
            drain(klast + jnp.int32(1), 1)
            return carry

        lax.fori_loop(jnp.int32(0), jnp.int32(NUM_LEVELS), level_body,
                      jnp.int32(0), unroll=False)

    return hash_enc


def _tbody(in_ref, out_ref):
    out_ref[...] = in_ref[...].T


@functools.lru_cache(maxsize=None)
def _transposer(n):
    B = 2048
    return pl.pallas_call(
        _tbody,
        grid=(n // B,),
        in_specs=[pl.BlockSpec((NUM_LEVELS * F, B),
                               lambda i: (jnp.int32(0), i))],
        out_specs=pl.BlockSpec((B, NUM_LEVELS * F),
                              lambda i: (i, jnp.int32(0))),
        out_shape=jax.ShapeDtypeStruct((n, NUM_LEVELS * F), jnp.float32),
    )


def kernel(x, tables):
    shape = x.shape
    xf = x.reshape(-1, 3)
    N = xf.shape[0]
    tbl = tables.reshape(NUM_LEVELS, HSIZE // 128, 128, F) \
        .transpose(0, 1, 3, 2).reshape(-1)
    rconst = jnp.asarray(
        np.repeat(np.array(_RES, np.float32), L))
    olm = _build(N)(xf[:, 0], xf[:, 1], xf[:, 2], tbl, rconst)
    out = _transposer(N)(olm)
    return out.reshape(*shape[:-1], NUM_LEVELS * F)
